# Initial kernel scaffold; baseline (speedup 1.0000x reference)
import jax
import jax.numpy as jnp
from jax.experimental import pallas as pl


def kernel(x, edge_index, W):
    raise NotImplementedError("write your pallas kernel here")


# speedup vs baseline: 3.5860x; 3.5860x over previous
import functools

import jax
import jax.numpy as jnp
from jax import lax
from jax.experimental import pallas as pl
from jax.experimental.pallas import tpu as pltpu
from jax.experimental.pallas import tpu_sc as plsc

NC = 2
NS = 16
NW = NC * NS
CHUNK = 128


def _spmm_partials(h, src, dst, n_acc, ept):
    n, d = h.shape
    opt = (n // NS) & ~7
    tail = n - opt * NS
    zrows = n_acc // NS
    n_chunks = ept // CHUNK
    mesh = plsc.VectorSubcoreMesh(core_axis_name="c", subcore_axis_name="s")

    @functools.partial(
        pl.kernel,
        mesh=mesh,
        out_type=[
            jax.ShapeDtypeStruct((n, d), jnp.float32),
            jax.ShapeDtypeStruct((n, d), jnp.float32),
        ],
        scratch_types=[
            pltpu.VMEM_SHARED((n_acc, d), jnp.float32),
            pltpu.VMEM((CHUNK,), jnp.int32),
            pltpu.VMEM((CHUNK,), jnp.int32),
            pltpu.VMEM((CHUNK, d), jnp.float32),
            pltpu.VMEM((CHUNK, d), jnp.float32),
            pltpu.SemaphoreType.DMA,
        ],
    )
    def hop(h_hbm, src_hbm, dst_hbm, p0_hbm, p1_hbm,
            acc, src_idx, dst_idx, rows, zbuf, sem):
        cid = lax.axis_index("c")
        sid = lax.axis_index("s")
        wid = sid * NC + cid

        def zrow(r, carry):
            for j in range(d // 16):
                zbuf[r, pl.ds(j * 16, 16)] = jnp.zeros((16,), jnp.float32)
            return carry
        lax.fori_loop(0, CHUNK, zrow, 0)
        zbase = sid * zrows
        nfull = zrows // CHUNK
        for b in range(nfull):
            pltpu.sync_copy(zbuf, acc.at[pl.ds(zbase + b * CHUNK, CHUNK)])
        rem = zrows - nfull * CHUNK
        if rem:
            pltpu.sync_copy(zbuf.at[pl.ds(0, rem)],
                            acc.at[pl.ds(zbase + nfull * CHUNK, rem)])
        plsc.subcore_barrier()

        ebase = wid * ept

        def echunk(i, carry):
            off = pl.multiple_of(ebase + i * CHUNK, CHUNK)
            pltpu.sync_copy(src_hbm.at[pl.ds(off, CHUNK)], src_idx)
            pltpu.sync_copy(dst_hbm.at[pl.ds(off, CHUNK)], dst_idx)
            pltpu.async_copy(h_hbm.at[src_idx], rows, sem).wait()
            pltpu.sync_copy(rows, acc.at[dst_idx], add=True)
            return carry
        lax.fori_loop(0, n_chunks, echunk, 0)
        plsc.subcore_barrier()

        obase = pl.multiple_of(sid * opt, 8)
        p_hbm = [p0_hbm, p1_hbm]
        for c in range(NC):
            @pl.when(cid == c)
            def _(c=c):
                pltpu.sync_copy(acc.at[pl.ds(obase, opt)],
                                p_hbm[c].at[pl.ds(obase, opt)])
                if tail:
                    @pl.when(sid == 0)
                    def _():
                        pltpu.sync_copy(
                            acc.at[pl.ds(opt * NS, tail)],
                            p_hbm[c].at[pl.ds(opt * NS, tail)])

    return hop(h, src, dst)


def _hop_combine(p0, p1, acc, w):
    n, d = p0.shape
    bn = 1000
    grid = (n // bn,)
    row_spec = pl.BlockSpec((bn, d), lambda i: (i, 0))
    w_spec = pl.BlockSpec((1, d), lambda i: (0, 0))

    def body(p0_ref, p1_ref, acc_ref, w_ref, h_ref, out_ref):
        h = p0_ref[...] + p1_ref[...]
        s = jnp.sum(h * w_ref[...], axis=1, keepdims=True)
        h_ref[...] = h
        out_ref[...] = acc_ref[...] + s * h

    return pl.pallas_call(
        body,
        grid=grid,
        in_specs=[row_spec, row_spec, row_spec, w_spec],
        out_specs=[row_spec, row_spec],
        out_shape=[
            jax.ShapeDtypeStruct((n, d), jnp.float32),
            jax.ShapeDtypeStruct((n, d), jnp.float32),
        ],
    )(p0, p1, acc, w)


def _init_acc(x, w):
    n, d = x.shape
    bn = 1000
    row_spec = pl.BlockSpec((bn, d), lambda i: (i, 0))
    w_spec = pl.BlockSpec((1, d), lambda i: (0, 0))

    def body(x_ref, w_ref, out_ref):
        xv = x_ref[...]
        s = jnp.sum(xv * w_ref[...], axis=1, keepdims=True)
        out_ref[...] = s * xv

    return pl.pallas_call(
        body,
        grid=(n // bn,),
        in_specs=[row_spec, w_spec],
        out_specs=row_spec,
        out_shape=jax.ShapeDtypeStruct((n, d), jnp.float32),
    )(x, w)


def kernel(x, edge_index, W):
    n, d = x.shape
    e = edge_index.shape[1]
    k_hops = 10

    ept = -(-e // (NW * CHUNK)) * CHUNK
    e_pad = ept * NW
    n_acc = -(-(n + 8) // (NS * CHUNK)) * NS * CHUNK
    src = edge_index[0].astype(jnp.int32)
    dst = edge_index[1].astype(jnp.int32)
    pad = e_pad - e
    src = jnp.concatenate([src, jnp.zeros((pad,), jnp.int32)])
    dst = jnp.concatenate([dst, jnp.full((pad,), n_acc - 8, jnp.int32)])

    acc = _init_acc(x, W)
    h = x
    for _ in range(k_hops):
        p0, p1 = _spmm_partials(h, src, dst, n_acc, ept)
        h, acc = _hop_combine(p0, p1, acc, W)
    return acc
